# baseline (speedup 1.0000x reference)
import functools

import jax
import jax.numpy as jnp
from jax import lax
from jax.experimental import pallas as pl
from jax.experimental.pallas import tpu as pltpu
from jax.experimental.pallas import tpu_sc as plsc

S, B, D = 4096, 2, 1024
L = 16
NCORES, NSUB = 2, 16
NW = NCORES * NSUB
P = 16
POS_PER_W = S // NW
CHUNKS = POS_PER_W // P
NBUF = 2
G = 8

_mesh = plsc.VectorSubcoreMesh(core_axis_name="c", subcore_axis_name="s")


@functools.partial(
    pl.kernel,
    out_type=jax.ShapeDtypeStruct((S, B, D), jnp.float32),
    mesh=_mesh,
    scratch_types=[
        pltpu.VMEM((NBUF, P, B, D), jnp.float32),
        pltpu.VMEM((NBUF, P, D), jnp.float32),
        pltpu.SemaphoreType.DMA((NBUF,)),
        pltpu.SemaphoreType.DMA((NBUF,)),
        pltpu.SemaphoreType.DMA((NBUF,)),
    ],
)
def _sc_add(x_hbm, pe_hbm, out_hbm, xb, peb, sx, sp, so):
    wid = lax.axis_index("s") * NCORES + lax.axis_index("c")
    pos0 = wid * POS_PER_W

    def in_copies(g, slot):
        sl = pl.ds(pos0 + g * P, P)
        return (pltpu.make_async_copy(x_hbm.at[sl], xb.at[slot], sx.at[slot]),
                pltpu.make_async_copy(pe_hbm.at[sl], peb.at[slot], sp.at[slot]))

    def out_copy(g, slot):
        return pltpu.make_async_copy(
            xb.at[slot], out_hbm.at[pl.ds(pos0 + g * P, P)], so.at[slot])

    def compute(slot):
        def body(p, carry):
            for j0 in range(0, D // L, G):
                sls = [pl.ds((j0 + j) * L, L) for j in range(G)]
                pvs = [peb[slot, p, dsl] for dsl in sls]
                a0 = [xb[slot, p, 0, dsl] + pv for dsl, pv in zip(sls, pvs)]
                a1 = [xb[slot, p, 1, dsl] + pv for dsl, pv in zip(sls, pvs)]
                for dsl, v in zip(sls, a0):
                    xb[slot, p, 0, dsl] = v
                for dsl, v in zip(sls, a1):
                    xb[slot, p, 1, dsl] = v
            return carry
        lax.fori_loop(0, P, body, 0)

    for g0 in range(NBUF - 1):
        cx, cp = in_copies(g0, g0)
        cx.start()
        cp.start()

    def chunk_body(g, carry):
        slot = lax.rem(g, NBUF)

        @pl.when(g + NBUF - 1 < CHUNKS)
        def _():
            @pl.when(g >= 1)
            def _():
                out_copy(g - 1, lax.rem(g - 1, NBUF)).wait()
            nslot = lax.rem(g + NBUF - 1, NBUF)
            cx, cp = in_copies(g + NBUF - 1, nslot)
            cx.start()
            cp.start()

        cx, cp = in_copies(g, slot)
        cx.wait()
        cp.wait()
        compute(slot)
        out_copy(g, slot).start()
        return carry

    lax.fori_loop(0, CHUNKS, chunk_body, 0)
    for g in range(CHUNKS - NBUF, CHUNKS):
        out_copy(g, g % NBUF).wait()


def kernel(x, pe):
    return _sc_add(x, pe)


# speedup vs baseline: 1.0779x; 1.0779x over previous
import functools

import jax
import jax.numpy as jnp
from jax import lax
from jax.experimental import pallas as pl
from jax.experimental.pallas import tpu as pltpu
from jax.experimental.pallas import tpu_sc as plsc

S, B, D = 4096, 2, 1024
L = 16
NCORES, NSUB = 2, 16
NW = NCORES * NSUB
P = 4
POS_PER_W = S // NW
CHUNKS = POS_PER_W // P
NBUF = 8
G = 8

_mesh = plsc.VectorSubcoreMesh(core_axis_name="c", subcore_axis_name="s")


@functools.partial(
    pl.kernel,
    out_type=jax.ShapeDtypeStruct((S, B, D), jnp.float32),
    mesh=_mesh,
    scratch_types=[
        pltpu.VMEM((NBUF, P, B, D), jnp.float32),
        pltpu.VMEM((NBUF, P, D), jnp.float32),
        pltpu.SemaphoreType.DMA((NBUF,)),
        pltpu.SemaphoreType.DMA((NBUF,)),
        pltpu.SemaphoreType.DMA((NBUF,)),
    ],
)
def _sc_add(x_hbm, pe_hbm, out_hbm, xb, peb, sx, sp, so):
    wid = lax.axis_index("s") * NCORES + lax.axis_index("c")
    pos0 = wid * POS_PER_W

    def in_copies(g, slot):
        sl = pl.ds(pos0 + g * P, P)
        return (pltpu.make_async_copy(x_hbm.at[sl], xb.at[slot], sx.at[slot]),
                pltpu.make_async_copy(pe_hbm.at[sl], peb.at[slot], sp.at[slot]))

    def out_copy(g, slot):
        return pltpu.make_async_copy(
            xb.at[slot], out_hbm.at[pl.ds(pos0 + g * P, P)], so.at[slot])

    def compute(slot):
        def body(p, carry):
            for j0 in range(0, D // L, G):
                sls = [pl.ds((j0 + j) * L, L) for j in range(G)]
                pvs = [peb[slot, p, dsl] for dsl in sls]
                a0 = [xb[slot, p, 0, dsl] + pv for dsl, pv in zip(sls, pvs)]
                a1 = [xb[slot, p, 1, dsl] + pv for dsl, pv in zip(sls, pvs)]
                for dsl, v in zip(sls, a0):
                    xb[slot, p, 0, dsl] = v
                for dsl, v in zip(sls, a1):
                    xb[slot, p, 1, dsl] = v
            return carry
        lax.fori_loop(0, P, body, 0)

    for g0 in range(NBUF - 1):
        cx, cp = in_copies(g0, g0)
        cx.start()
        cp.start()

    def chunk_body(g, carry):
        slot = lax.rem(g, NBUF)

        @pl.when(g + NBUF - 1 < CHUNKS)
        def _():
            @pl.when(g >= 1)
            def _():
                out_copy(g - 1, lax.rem(g - 1, NBUF)).wait()
            nslot = lax.rem(g + NBUF - 1, NBUF)
            cx, cp = in_copies(g + NBUF - 1, nslot)
            cx.start()
            cp.start()

        cx, cp = in_copies(g, slot)
        cx.wait()
        cp.wait()
        compute(slot)
        out_copy(g, slot).start()
        return carry

    lax.fori_loop(0, CHUNKS, chunk_body, 0)
    for g in range(CHUNKS - NBUF, CHUNKS):
        out_copy(g, g % NBUF).wait()


def kernel(x, pe):
    return _sc_add(x, pe)
